# baseline (speedup 1.0000x reference)
import functools

import jax
import jax.numpy as jnp
from jax import lax
from jax.experimental import pallas as pl
from jax.experimental.pallas import tpu as pltpu
from jax.experimental.pallas import tpu_sc as plsc


def _pooled_sum_sc(ids_flat, embed_table, B, S):
    V, D = embed_table.shape
    NW = 32
    b_per_w = B // NW
    L = 16
    nchunk = D // L
    C0 = 128
    C1 = S - C0
    U = 8

    mesh = plsc.VectorSubcoreMesh(core_axis_name="c", subcore_axis_name="s")

    @functools.partial(
        pl.kernel,
        out_type=jax.ShapeDtypeStruct((B, D), jnp.float32),
        mesh=mesh,
        scratch_types=[
            pltpu.VMEM((b_per_w * S,), jnp.int32),
            pltpu.VMEM((2, S, D), jnp.float32),
            pltpu.VMEM((b_per_w, D), jnp.float32),
            pltpu.SemaphoreType.DMA((2,)),
        ],
        compiler_params=pltpu.CompilerParams(use_tc_tiling_on_sc=False),
    )
    def pooled(ids_hbm, table_hbm, out_hbm, idx_all, rows_v, out_all, sems):
        wid = lax.axis_index("s") * 2 + lax.axis_index("c")
        base = wid * b_per_w

        pltpu.sync_copy(ids_hbm.at[pl.ds(base * S, b_per_w * S)], idx_all)

        def fire(buf, row):
            @pl.when(row < b_per_w)
            def _():
                pltpu.async_copy(
                    table_hbm.at[idx_all.at[pl.ds(row * S, C0)]],
                    rows_v.at[buf, pl.ds(0, C0)], sems.at[buf])
                pltpu.async_copy(
                    table_hbm.at[idx_all.at[pl.ds(row * S + C0, C1)]],
                    rows_v.at[buf, pl.ds(C0, C1)], sems.at[buf])

        def drain(buf):
            pltpu.make_async_copy(
                table_hbm.at[pl.ds(0, S)], rows_v.at[buf], sems.at[buf]
            ).wait()

        fire(0, jnp.int32(0))
        fire(1, jnp.int32(1))

        def pair_body(p, carry):
            for k in range(2):
                row = p * 2 + k
                drain(k)

                def red(t, accs):
                    s0 = t * U
                    out = list(accs)
                    for u in range(U):
                        for c in range(nchunk):
                            out[c] = out[c] + rows_v[k, s0 + u, pl.ds(c * L, L)]
                    return tuple(out)

                zeros = tuple(
                    jnp.zeros((L,), jnp.float32) for _ in range(nchunk))
                accs = lax.fori_loop(0, S // U, red, zeros)
                fire(k, row + 2)
                for c in range(nchunk):
                    out_all[row, pl.ds(c * L, L)] = accs[c]
            return carry

        lax.fori_loop(0, b_per_w // 2, pair_body, 0)
        pltpu.sync_copy(out_all, out_hbm.at[pl.ds(base, b_per_w)])

    return pooled(ids_flat, embed_table)


def _linear_tc(pooled, Ws, b2d):
    B, D = pooled.shape
    C = Ws.shape[1]
    BM = 512

    def mm(x_ref, w_ref, b_ref, o_ref):
        o_ref[...] = (
            jnp.dot(x_ref[...], w_ref[...], preferred_element_type=jnp.float32)
            + b_ref[...]
        )

    return pl.pallas_call(
        mm,
        grid=(B // BM,),
        in_specs=[
            pl.BlockSpec((BM, D), lambda i: (i, 0)),
            pl.BlockSpec((D, C), lambda i: (0, 0)),
            pl.BlockSpec((1, C), lambda i: (0, 0)),
        ],
        out_specs=pl.BlockSpec((BM, C), lambda i: (i, 0)),
        out_shape=jax.ShapeDtypeStruct((B, C), jnp.float32),
    )(pooled, Ws, b2d)


@jax.jit
def kernel(input_ids, embed_table, W, b):
    B, S = input_ids.shape
    pooled = _pooled_sum_sc(
        input_ids.astype(jnp.int32).reshape(-1), embed_table, B, S)
    Ws = W * (1.0 / S)
    b2d = b.reshape(1, -1)
    return _linear_tc(pooled, Ws, b2d)


# speedup vs baseline: 4.1153x; 4.1153x over previous
import functools

import jax
import jax.numpy as jnp
from jax import lax
from jax.experimental import pallas as pl
from jax.experimental.pallas import tpu as pltpu
from jax.experimental.pallas import tpu_sc as plsc


def _project_table_tc(tableT, W2, bb):
    C, D = W2.shape
    V = tableT.shape[1]
    NBLK = 16384
    grid = pl.cdiv(V, NBLK)

    def proj(t_ref, w_ref, b_ref, o0_ref, o1_ref):
        r = (
            jnp.dot(w_ref[...], t_ref[...], preferred_element_type=jnp.float32)
            + b_ref[...]
        )
        o0_ref[...] = r[0]
        o1_ref[...] = r[1]

    return pl.pallas_call(
        proj,
        grid=(grid,),
        in_specs=[
            pl.BlockSpec((D, NBLK), lambda i: (0, i)),
            pl.BlockSpec((C, D), lambda i: (0, 0)),
            pl.BlockSpec((C, 1), lambda i: (0, 0)),
        ],
        out_specs=[
            pl.BlockSpec((NBLK,), lambda i: (i,)),
            pl.BlockSpec((NBLK,), lambda i: (i,)),
        ],
        out_shape=[
            jax.ShapeDtypeStruct((V,), jnp.float32),
            jax.ShapeDtypeStruct((V,), jnp.float32),
        ],
    )(tableT, W2, bb)


def _gather_pool_sc(idsT, p0, p1, B, S):
    NW = 32
    b_per_w = B // NW
    L = 16

    mesh = plsc.VectorSubcoreMesh(core_axis_name="c", subcore_axis_name="s")

    @functools.partial(
        pl.kernel,
        out_type=jax.ShapeDtypeStruct((2, B), jnp.float32),
        mesh=mesh,
        scratch_types=[
            pltpu.VMEM((S, b_per_w), jnp.int32),
            pltpu.VMEM((S * b_per_w,), jnp.float32),
            pltpu.VMEM((S * b_per_w,), jnp.float32),
            pltpu.VMEM((2, b_per_w), jnp.float32),
            pltpu.SemaphoreType.DMA,
        ],
        compiler_params=pltpu.CompilerParams(use_tc_tiling_on_sc=False),
    )
    def gpool(ids_hbm, p0_hbm, p1_hbm, out_hbm, idx_v, g0_v, g1_v, out_v, sem):
        wid = lax.axis_index("s") * 2 + lax.axis_index("c")
        base = wid * b_per_w

        pltpu.sync_copy(ids_hbm.at[:, pl.ds(base, b_per_w)], idx_v)

        def fire(s, carry):
            pltpu.async_copy(
                p0_hbm.at[idx_v.at[s]],
                g0_v.at[pl.ds(s * b_per_w, b_per_w)], sem)
            pltpu.async_copy(
                p1_hbm.at[idx_v.at[s]],
                g1_v.at[pl.ds(s * b_per_w, b_per_w)], sem)
            return carry

        lax.fori_loop(0, S, fire, 0)

        pltpu.make_async_copy(
            p0_hbm.at[pl.ds(0, S * b_per_w)], g0_v, sem).wait()
        pltpu.make_async_copy(
            p1_hbm.at[pl.ds(0, S * b_per_w)], g1_v, sem).wait()

        for jg in range(b_per_w // L):
            off = jg * L

            def red(s, accs):
                a0, a1 = accs
                a0 = a0 + g0_v[pl.ds(s * b_per_w + off, L)]
                a1 = a1 + g1_v[pl.ds(s * b_per_w + off, L)]
                return (a0, a1)

            z = jnp.zeros((L,), jnp.float32)
            a0, a1 = lax.fori_loop(0, S, red, (z, z))
            out_v[0, pl.ds(off, L)] = a0
            out_v[1, pl.ds(off, L)] = a1

        pltpu.sync_copy(out_v.at[0], out_hbm.at[0, pl.ds(base, b_per_w)])
        pltpu.sync_copy(out_v.at[1], out_hbm.at[1, pl.ds(base, b_per_w)])

    return gpool(idsT, p0, p1)


@jax.jit
def kernel(input_ids, embed_table, W, b):
    B, S = input_ids.shape
    C = W.shape[1]
    idsT = input_ids.astype(jnp.int32).T
    tableT = embed_table.T
    W2 = (W * (1.0 / S)).T
    bb = (b * (1.0 / S)).reshape(C, 1)
    p0, p1 = _project_table_tc(tableT, W2, bb)
    pooledT = _gather_pool_sc(idsT, p0, p1, B, S)
    return pooledT.T


# speedup vs baseline: 4.3123x; 1.0479x over previous
import functools

import jax
import jax.numpy as jnp
from jax import lax
from jax.experimental import pallas as pl
from jax.experimental.pallas import tpu as pltpu
from jax.experimental.pallas import tpu_sc as plsc


def _project_table_tc(tableT, W2, bb):
    C, D = W2.shape
    V = tableT.shape[1]
    NBLK = 32768
    grid = pl.cdiv(V, NBLK)

    def proj(t_ref, w_ref, b_ref, o0_ref, o1_ref):
        r = (
            jnp.dot(w_ref[...], t_ref[...], preferred_element_type=jnp.float32)
            + b_ref[...]
        )
        o0_ref[...] = r[0]
        o1_ref[...] = r[1]

    return pl.pallas_call(
        proj,
        grid=(grid,),
        in_specs=[
            pl.BlockSpec((D, NBLK), lambda i: (0, i)),
            pl.BlockSpec((C, D), lambda i: (0, 0)),
            pl.BlockSpec((C, 1), lambda i: (0, 0)),
        ],
        out_specs=[
            pl.BlockSpec((NBLK,), lambda i: (i,)),
            pl.BlockSpec((NBLK,), lambda i: (i,)),
        ],
        out_shape=[
            jax.ShapeDtypeStruct((V,), jnp.float32),
            jax.ShapeDtypeStruct((V,), jnp.float32),
        ],
    )(tableT, W2, bb)


def _gather_pool_sc(idsT, p0, p1, B, S):
    NW = 32
    b_per_w = B // NW
    L = 16

    mesh = plsc.VectorSubcoreMesh(core_axis_name="c", subcore_axis_name="s")

    @functools.partial(
        pl.kernel,
        out_type=jax.ShapeDtypeStruct((2, B), jnp.float32),
        mesh=mesh,
        scratch_types=[
            pltpu.VMEM((S, b_per_w), jnp.int32),
            pltpu.VMEM((S * b_per_w,), jnp.float32),
            pltpu.VMEM((S * b_per_w,), jnp.float32),
            pltpu.VMEM((2, b_per_w), jnp.float32),
            pltpu.SemaphoreType.DMA,
        ],
        compiler_params=pltpu.CompilerParams(use_tc_tiling_on_sc=False),
    )
    def gpool(ids_hbm, p0_hbm, p1_hbm, out_hbm, idx_v, g0_v, g1_v, out_v, sem):
        wid = lax.axis_index("s") * 2 + lax.axis_index("c")
        base = wid * b_per_w

        pltpu.sync_copy(ids_hbm.at[:, pl.ds(base, b_per_w)], idx_v)

        def fire(s, carry):
            pltpu.async_copy(
                p0_hbm.at[idx_v.at[s]],
                g0_v.at[pl.ds(s * b_per_w, b_per_w)], sem)
            pltpu.async_copy(
                p1_hbm.at[idx_v.at[s]],
                g1_v.at[pl.ds(s * b_per_w, b_per_w)], sem)
            return carry

        lax.fori_loop(0, S, fire, 0)

        pltpu.make_async_copy(
            p0_hbm.at[pl.ds(0, S * b_per_w)], g0_v, sem).wait()
        pltpu.make_async_copy(
            p1_hbm.at[pl.ds(0, S * b_per_w)], g1_v, sem).wait()

        for jg in range(b_per_w // L):
            off = jg * L

            def red(s, accs):
                a0, a1 = accs
                a0 = a0 + g0_v[pl.ds(s * b_per_w + off, L)]
                a1 = a1 + g1_v[pl.ds(s * b_per_w + off, L)]
                return (a0, a1)

            z = jnp.zeros((L,), jnp.float32)
            a0, a1 = lax.fori_loop(0, S, red, (z, z))
            out_v[0, pl.ds(off, L)] = a0
            out_v[1, pl.ds(off, L)] = a1

        pltpu.sync_copy(out_v.at[0], out_hbm.at[0, pl.ds(base, b_per_w)])
        pltpu.sync_copy(out_v.at[1], out_hbm.at[1, pl.ds(base, b_per_w)])

    return gpool(idsT, p0, p1)


@jax.jit
def kernel(input_ids, embed_table, W, b):
    B, S = input_ids.shape
    C = W.shape[1]
    idsT = input_ids.astype(jnp.int32).T
    tableT = embed_table.T
    W2 = (W * (1.0 / S)).T
    bb = (b * (1.0 / S)).reshape(C, 1)
    p0, p1 = _project_table_tc(tableT, W2, bb)
    pooledT = _gather_pool_sc(idsT, p0, p1, B, S)
    return pooledT.T


# speedup vs baseline: 4.3333x; 1.0049x over previous
import functools

import jax
import jax.numpy as jnp
from jax import lax
from jax.experimental import pallas as pl
from jax.experimental.pallas import tpu as pltpu
from jax.experimental.pallas import tpu_sc as plsc


def _project_table_tc(tableT, W2, bb):
    C, D = W2.shape
    V = tableT.shape[1]
    NBLK = 65536
    grid = pl.cdiv(V, NBLK)

    def proj(t_ref, w_ref, b_ref, o0_ref, o1_ref):
        r = (
            jnp.dot(w_ref[...], t_ref[...], preferred_element_type=jnp.float32)
            + b_ref[...]
        )
        o0_ref[...] = r[0]
        o1_ref[...] = r[1]

    return pl.pallas_call(
        proj,
        grid=(grid,),
        in_specs=[
            pl.BlockSpec((D, NBLK), lambda i: (0, i)),
            pl.BlockSpec((C, D), lambda i: (0, 0)),
            pl.BlockSpec((C, 1), lambda i: (0, 0)),
        ],
        out_specs=[
            pl.BlockSpec((NBLK,), lambda i: (i,)),
            pl.BlockSpec((NBLK,), lambda i: (i,)),
        ],
        out_shape=[
            jax.ShapeDtypeStruct((V,), jnp.float32),
            jax.ShapeDtypeStruct((V,), jnp.float32),
        ],
    )(tableT, W2, bb)


def _gather_pool_sc(idsT, p0, p1, B, S):
    NW = 32
    b_per_w = B // NW
    L = 16

    mesh = plsc.VectorSubcoreMesh(core_axis_name="c", subcore_axis_name="s")

    @functools.partial(
        pl.kernel,
        out_type=jax.ShapeDtypeStruct((2, B), jnp.float32),
        mesh=mesh,
        scratch_types=[
            pltpu.VMEM((S, b_per_w), jnp.int32),
            pltpu.VMEM((S * b_per_w,), jnp.float32),
            pltpu.VMEM((S * b_per_w,), jnp.float32),
            pltpu.VMEM((2, b_per_w), jnp.float32),
            pltpu.SemaphoreType.DMA,
        ],
        compiler_params=pltpu.CompilerParams(use_tc_tiling_on_sc=False),
    )
    def gpool(ids_hbm, p0_hbm, p1_hbm, out_hbm, idx_v, g0_v, g1_v, out_v, sem):
        wid = lax.axis_index("s") * 2 + lax.axis_index("c")
        base = wid * b_per_w

        pltpu.sync_copy(ids_hbm.at[:, pl.ds(base, b_per_w)], idx_v)

        def fire(s, carry):
            pltpu.async_copy(
                p0_hbm.at[idx_v.at[s]],
                g0_v.at[pl.ds(s * b_per_w, b_per_w)], sem)
            pltpu.async_copy(
                p1_hbm.at[idx_v.at[s]],
                g1_v.at[pl.ds(s * b_per_w, b_per_w)], sem)
            return carry

        lax.fori_loop(0, S, fire, 0)

        pltpu.make_async_copy(
            p0_hbm.at[pl.ds(0, S * b_per_w)], g0_v, sem).wait()
        pltpu.make_async_copy(
            p1_hbm.at[pl.ds(0, S * b_per_w)], g1_v, sem).wait()

        for jg in range(b_per_w // L):
            off = jg * L

            def red(s, accs):
                a0, a1 = accs
                a0 = a0 + g0_v[pl.ds(s * b_per_w + off, L)]
                a1 = a1 + g1_v[pl.ds(s * b_per_w + off, L)]
                return (a0, a1)

            z = jnp.zeros((L,), jnp.float32)
            a0, a1 = lax.fori_loop(0, S, red, (z, z))
            out_v[0, pl.ds(off, L)] = a0
            out_v[1, pl.ds(off, L)] = a1

        pltpu.sync_copy(out_v.at[0], out_hbm.at[0, pl.ds(base, b_per_w)])
        pltpu.sync_copy(out_v.at[1], out_hbm.at[1, pl.ds(base, b_per_w)])

    return gpool(idsT, p0, p1)


@jax.jit
def kernel(input_ids, embed_table, W, b):
    B, S = input_ids.shape
    C = W.shape[1]
    idsT = input_ids.astype(jnp.int32).T
    tableT = embed_table.T
    W2 = (W * (1.0 / S)).T
    bb = (b * (1.0 / S)).reshape(C, 1)
    p0, p1 = _project_table_tc(tableT, W2, bb)
    pooledT = _gather_pool_sc(idsT, p0, p1, B, S)
    return pooledT.T
